# initial kernel scaffold (unmeasured)
import jax
import jax.numpy as jnp
from jax import lax
from jax.experimental import pallas as pl
from jax.experimental.pallas import tpu as pltpu

N_DEV = 32
BLK = 64


def kernel(x, Wq, K_ext, V_ext, Wo):
    B, S, DM = x.shape
    Hq, Dh = K_ext.shape[2], K_ext.shape[3]
    HD = Hq * Dh
    Sg = N_DEV * S

    K2 = K_ext.reshape(B, S, HD)
    V2 = V_ext.reshape(B, S, HD)

    def body(x_ref, wq_ref, k_ref, v_ref, wo_ref, out_ref,
             kv_full, send_sems, recv_sems):
        my = lax.axis_index("i")
        left = lax.rem(my - 1 + N_DEV, N_DEV)
        right = lax.rem(my + 1, N_DEV)

        kv_full[pl.ds(my, 1), 0] = k_ref[...].astype(jnp.bfloat16)[None]
        kv_full[pl.ds(my, 1), 1] = v_ref[...].astype(jnp.bfloat16)[None]

        barrier_sem = pltpu.get_barrier_semaphore()
        for nbr in (left, right):
            pl.semaphore_signal(
                barrier_sem, inc=1,
                device_id=(nbr,), device_id_type=pl.DeviceIdType.MESH,
            )
        pl.semaphore_wait(barrier_sem, 2)

        for h in range(N_DEV - 1):
            s = lax.rem(my - h + N_DEV, N_DEV)
            rdma = pltpu.make_async_remote_copy(
                src_ref=kv_full.at[s],
                dst_ref=kv_full.at[s],
                send_sem=send_sems.at[h],
                recv_sem=recv_sems.at[h],
                device_id=(right,),
                device_id_type=pl.DeviceIdType.MESH,
            )
            rdma.start()
            rdma.wait()

        wq = wq_ref[...].astype(jnp.bfloat16)
        wo = wo_ref[...].astype(jnp.bfloat16)

        rowb = (my * S + lax.broadcasted_iota(jnp.int32, (S, Sg), 0)) // BLK
        colb = lax.broadcasted_iota(jnp.int32, (S, Sg), 1) // BLK
        mask = colb <= rowb

        for b in range(B):
            q_b = jnp.dot(x_ref[b].astype(jnp.bfloat16), wq,
                          preferred_element_type=jnp.float32)
            ctxs = []
            for h in range(Hq):
                qh = q_b[:, h * Dh:(h + 1) * Dh].astype(jnp.bfloat16)
                kh = kv_full[:, 0, b, :, h * Dh:(h + 1) * Dh].reshape(Sg, Dh)
                vh = kv_full[:, 1, b, :, h * Dh:(h + 1) * Dh].reshape(Sg, Dh)
                sc = lax.dot_general(
                    qh, kh, (((1,), (1,)), ((), ())),
                    preferred_element_type=jnp.float32) * 0.125
                sc = jnp.where(mask, sc, -1e9)
                m = jnp.max(sc, axis=-1, keepdims=True)
                w = jnp.exp(sc - m)
                den = jnp.sum(w, axis=-1, keepdims=True)
                w = (w / den).astype(jnp.bfloat16)
                ctxs.append(jnp.dot(w, vh, preferred_element_type=jnp.float32))
            ctx = jnp.concatenate(ctxs, axis=1)
            out_ref[b] = jnp.dot(ctx.astype(jnp.bfloat16), wo,
                                 preferred_element_type=jnp.float32)

    return pl.pallas_call(
        body,
        out_shape=jax.ShapeDtypeStruct((B, S, DM), jnp.float32),
        in_specs=[pl.BlockSpec(memory_space=pltpu.VMEM)] * 5,
        out_specs=pl.BlockSpec(memory_space=pltpu.VMEM),
        scratch_shapes=[
            pltpu.VMEM((N_DEV, 2, B, S, HD), jnp.bfloat16),
            pltpu.SemaphoreType.DMA((N_DEV - 1,)),
            pltpu.SemaphoreType.DMA((N_DEV - 1,)),
        ],
        compiler_params=pltpu.CompilerParams(collective_id=0),
    )(x, Wq, K2, V2, Wo)


# baseline (device time: 285495 ns/iter reference)
import jax
import jax.numpy as jnp
from jax import lax
from jax.experimental import pallas as pl
from jax.experimental.pallas import tpu as pltpu

N_DEV = 32
BLK = 64


def kernel(x, Wq, K_ext, V_ext, Wo):
    B, S, DM = x.shape
    Hq, Dh = K_ext.shape[2], K_ext.shape[3]
    HD = Hq * Dh
    Sg = N_DEV * S

    K2 = K_ext.reshape(B, S, HD)
    V2 = V_ext.reshape(B, S, HD)

    def body(x_ref, wq_ref, k_ref, v_ref, wo_ref, out_ref,
             kv_full, send_sems, recv_sems):
        my = lax.axis_index("i")
        left = lax.rem(my - 1 + N_DEV, N_DEV)
        right = lax.rem(my + 1, N_DEV)

        kv_full[pl.ds(my, 1), 0] = k_ref[...].astype(jnp.bfloat16)[None]
        kv_full[pl.ds(my, 1), 1] = v_ref[...].astype(jnp.bfloat16)[None]

        barrier_sem = pltpu.get_barrier_semaphore()
        for nbr in (left, right):
            pl.semaphore_signal(
                barrier_sem, inc=1,
                device_id=(nbr,), device_id_type=pl.DeviceIdType.MESH,
            )
        pl.semaphore_wait(barrier_sem, 2)

        for h in range(N_DEV - 1):
            s = lax.rem(my - h + N_DEV, N_DEV)
            rdma = pltpu.make_async_remote_copy(
                src_ref=kv_full.at[s],
                dst_ref=kv_full.at[s],
                send_sem=send_sems.at[h],
                recv_sem=recv_sems.at[h],
                device_id=(right,),
                device_id_type=pl.DeviceIdType.MESH,
            )
            rdma.start()
            rdma.wait()

        wq = wq_ref[...].astype(jnp.bfloat16)
        wo = wo_ref[...].astype(jnp.bfloat16)

        rowb = (my * S + lax.broadcasted_iota(jnp.int32, (S, Sg), 0)) // BLK
        colb = lax.broadcasted_iota(jnp.int32, (S, Sg), 1) // BLK
        mask = colb <= rowb

        for b in range(B):
            q_b = jnp.dot(x_ref[b].astype(jnp.bfloat16), wq,
                          preferred_element_type=jnp.float32)
            ctxs = []
            for h in range(Hq):
                qh = q_b[:, h * Dh:(h + 1) * Dh].astype(jnp.bfloat16)
                kh = kv_full[:, 0, b, :, h * Dh:(h + 1) * Dh].reshape(Sg, Dh)
                vh = kv_full[:, 1, b, :, h * Dh:(h + 1) * Dh].reshape(Sg, Dh)
                sc = lax.dot_general(
                    qh, kh, (((1,), (1,)), ((), ())),
                    preferred_element_type=jnp.float32) * 0.125
                sc = jnp.where(mask, sc, -1e9)
                m = jnp.max(sc, axis=-1, keepdims=True)
                w = jnp.exp(sc - m)
                den = jnp.sum(w, axis=-1, keepdims=True)
                w = (w / den).astype(jnp.bfloat16)
                ctxs.append(jnp.dot(w, vh, preferred_element_type=jnp.float32))
            ctx = jnp.concatenate(ctxs, axis=1)
            out_ref[b] = jnp.dot(ctx.astype(jnp.bfloat16), wo,
                                 preferred_element_type=jnp.float32)

    return pl.pallas_call(
        body,
        out_shape=jax.ShapeDtypeStruct((B, S, DM), jnp.float32),
        in_specs=[pl.BlockSpec(memory_space=pltpu.VMEM)] * 5,
        out_specs=pl.BlockSpec(memory_space=pltpu.VMEM),
        scratch_shapes=[
            pltpu.VMEM((N_DEV, 2, B, S, HD), jnp.bfloat16),
            pltpu.SemaphoreType.DMA((N_DEV - 1,)),
            pltpu.SemaphoreType.DMA((N_DEV - 1,)),
        ],
        compiler_params=pltpu.CompilerParams(
            collective_id=0,
            vmem_limit_bytes=100 * 1024 * 1024,
        ),
    )(x, Wq, K2, V2, Wo)


# device time: 255803 ns/iter; 1.1161x vs baseline; 1.1161x over previous
import jax
import jax.numpy as jnp
from jax import lax
from jax.experimental import pallas as pl
from jax.experimental.pallas import tpu as pltpu

N_DEV = 32
BLK = 64


def kernel(x, Wq, K_ext, V_ext, Wo):
    B, S, DM = x.shape
    Hq, Dh = K_ext.shape[2], K_ext.shape[3]
    HD = Hq * Dh
    Sg = N_DEV * S

    K2 = K_ext.reshape(B, S, HD)
    V2 = V_ext.reshape(B, S, HD)

    def body(x_ref, wq_ref, k_ref, v_ref, wo_ref, out_ref,
             kv_full, send_sems, recv_sems, send_sems_l, recv_sems_l):
        my = lax.axis_index("i")
        left = lax.rem(my - 1 + N_DEV, N_DEV)
        right = lax.rem(my + 1, N_DEV)

        kv_full[pl.ds(my, 1), 0] = k_ref[...].astype(jnp.bfloat16)[None]
        kv_full[pl.ds(my, 1), 1] = v_ref[...].astype(jnp.bfloat16)[None]

        barrier_sem = pltpu.get_barrier_semaphore()
        for nbr in (left, right):
            pl.semaphore_signal(
                barrier_sem, inc=1,
                device_id=(nbr,), device_id_type=pl.DeviceIdType.MESH,
            )
        pl.semaphore_wait(barrier_sem, 2)

        H_R = N_DEV // 2
        H_L = N_DEV // 2 - 1
        for h in range(H_R):
            s_r = lax.rem(my - h + N_DEV, N_DEV)
            rdma_r = pltpu.make_async_remote_copy(
                src_ref=kv_full.at[s_r],
                dst_ref=kv_full.at[s_r],
                send_sem=send_sems.at[h],
                recv_sem=recv_sems.at[h],
                device_id=(right,),
                device_id_type=pl.DeviceIdType.MESH,
            )
            rdma_r.start()
            if h < H_L:
                s_l = lax.rem(my + h, N_DEV)
                rdma_l = pltpu.make_async_remote_copy(
                    src_ref=kv_full.at[s_l],
                    dst_ref=kv_full.at[s_l],
                    send_sem=send_sems_l.at[h],
                    recv_sem=recv_sems_l.at[h],
                    device_id=(left,),
                    device_id_type=pl.DeviceIdType.MESH,
                )
                rdma_l.start()
                rdma_l.wait()
            rdma_r.wait()

        wq = wq_ref[...].astype(jnp.bfloat16)
        wo = wo_ref[...].astype(jnp.bfloat16)

        rowb = (my * S + lax.broadcasted_iota(jnp.int32, (S, Sg), 0)) // BLK
        colb = lax.broadcasted_iota(jnp.int32, (S, Sg), 1) // BLK
        mask = colb <= rowb

        for b in range(B):
            q_b = jnp.dot(x_ref[b].astype(jnp.bfloat16), wq,
                          preferred_element_type=jnp.float32)
            ctxs = []
            for h in range(Hq):
                qh = q_b[:, h * Dh:(h + 1) * Dh].astype(jnp.bfloat16)
                kh = kv_full[:, 0, b, :, h * Dh:(h + 1) * Dh].reshape(Sg, Dh)
                vh = kv_full[:, 1, b, :, h * Dh:(h + 1) * Dh].reshape(Sg, Dh)
                sc = lax.dot_general(
                    qh, kh, (((1,), (1,)), ((), ())),
                    preferred_element_type=jnp.float32) * 0.125
                sc = jnp.where(mask, sc, -1e9)
                m = jnp.max(sc, axis=-1, keepdims=True)
                w = jnp.exp(sc - m)
                den = jnp.sum(w, axis=-1, keepdims=True)
                w = (w / den).astype(jnp.bfloat16)
                ctxs.append(jnp.dot(w, vh, preferred_element_type=jnp.float32))
            ctx = jnp.concatenate(ctxs, axis=1)
            out_ref[b] = jnp.dot(ctx.astype(jnp.bfloat16), wo,
                                 preferred_element_type=jnp.float32)

    return pl.pallas_call(
        body,
        out_shape=jax.ShapeDtypeStruct((B, S, DM), jnp.float32),
        in_specs=[pl.BlockSpec(memory_space=pltpu.VMEM)] * 5,
        out_specs=pl.BlockSpec(memory_space=pltpu.VMEM),
        scratch_shapes=[
            pltpu.VMEM((N_DEV, 2, B, S, HD), jnp.bfloat16),
            pltpu.SemaphoreType.DMA((N_DEV // 2,)),
            pltpu.SemaphoreType.DMA((N_DEV // 2,)),
            pltpu.SemaphoreType.DMA((N_DEV // 2 - 1,)),
            pltpu.SemaphoreType.DMA((N_DEV // 2 - 1,)),
        ],
        compiler_params=pltpu.CompilerParams(
            collective_id=0,
            vmem_limit_bytes=100 * 1024 * 1024,
        ),
    )(x, Wq, K2, V2, Wo)


# device time: 200559 ns/iter; 1.4235x vs baseline; 1.2755x over previous
import jax
import jax.numpy as jnp
from jax import lax
from jax.experimental import pallas as pl
from jax.experimental.pallas import tpu as pltpu

N_DEV = 32
BLK = 64


def kernel(x, Wq, K_ext, V_ext, Wo):
    B, S, DM = x.shape
    Hq, Dh = K_ext.shape[2], K_ext.shape[3]
    HD = Hq * Dh
    SP = S + Hq

    K2 = K_ext.reshape(B, S, HD)
    V2 = V_ext.reshape(B, S, HD)

    def body(x_ref, wq_ref, k_ref, v_ref, wo_ref, out_ref,
             q_buf, p_in, p_out,
             q_send_sems, q_recv_sems, p_send_sems, p_recv_sems):
        my = lax.axis_index("i")
        bf = jnp.bfloat16

        wq = wq_ref[...].astype(bf)
        wo = wo_ref[...].astype(bf)

        for b in range(B):
            qb = jnp.dot(x_ref[b].astype(bf), wq,
                         preferred_element_type=jnp.float32) * 0.125
            q_buf[pl.ds(my, 1), b] = qb.astype(bf)[None]

        for jj in range(N_DEV - 1):
            @pl.when(jj < my)
            def _():
                rdma = pltpu.make_async_remote_copy(
                    src_ref=q_buf.at[my],
                    dst_ref=q_buf.at[my],
                    send_sem=q_send_sems.at[jj],
                    recv_sem=q_recv_sems.at[my],
                    device_id=(jj,),
                    device_id_type=pl.DeviceIdType.MESH,
                )
                rdma.start()

        rb = lax.broadcasted_iota(jnp.int32, (S, S), 0) // BLK
        cb = lax.broadcasted_iota(jnp.int32, (S, S), 1) // BLK
        mask0 = cb <= rb

        kl = [k_ref[b].astype(bf) for b in range(B)]
        vl = [v_ref[b].astype(bf) for b in range(B)]

        ctx0 = []
        s0 = []
        for b in range(B):
            ctx_h = []
            s_h = []
            for h in range(Hq):
                hs = slice(h * Dh, (h + 1) * Dh)
                q = q_buf[pl.ds(my, 1), b, :, hs][0]
                sc = lax.dot_general(q, kl[b][:, hs],
                                     (((1,), (1,)), ((), ())),
                                     preferred_element_type=jnp.float32)
                e = jnp.where(mask0, jnp.exp(sc), 0.0)
                s_h.append(jnp.sum(e, axis=1))
                ctx_h.append(jnp.dot(e.astype(bf), vl[b][:, hs],
                                     preferred_element_type=jnp.float32))
            ctx0.append(jnp.concatenate(ctx_h, axis=1))
            s0.append(jnp.stack(s_h, axis=0))
        ctx_acc0 = jnp.stack(ctx0, axis=0)
        s_acc0 = jnp.stack(s0, axis=0)

        def serve(i, carry):
            recv = pltpu.make_async_remote_copy(
                src_ref=q_buf.at[i], dst_ref=q_buf.at[i],
                send_sem=q_send_sems.at[0],
                recv_sem=q_recv_sems.at[i],
                device_id=(i,), device_id_type=pl.DeviceIdType.MESH,
            )
            recv.wait_recv()
            for b in range(B):
                for h in range(Hq):
                    hs = slice(h * Dh, (h + 1) * Dh)
                    q = q_buf[pl.ds(i, 1), b, :, hs][0]
                    sc = lax.dot_general(q, kl[b][:, hs],
                                         (((1,), (1,)), ((), ())),
                                         preferred_element_type=jnp.float32)
                    e = jnp.exp(sc)
                    p_out[pl.ds(i, 1), b, pl.ds(S + h, 1), :] = (
                        jnp.sum(e, axis=1).astype(bf)[None, None])
                    p_out[pl.ds(i, 1), b, :S, hs] = (
                        jnp.dot(e.astype(bf), vl[b][:, hs],
                                preferred_element_type=jnp.float32)
                        .astype(bf)[None])
            send = pltpu.make_async_remote_copy(
                src_ref=p_out.at[i],
                dst_ref=p_in.at[my],
                send_sem=p_send_sems.at[i],
                recv_sem=p_recv_sems.at[my],
                device_id=(i,), device_id_type=pl.DeviceIdType.MESH,
            )
            send.start()
            return carry

        lax.fori_loop(my + 1, N_DEV, serve, 0)

        def combine(j, carry):
            ctx_acc, s_acc = carry
            recv = pltpu.make_async_remote_copy(
                src_ref=p_in.at[j], dst_ref=p_in.at[j],
                send_sem=p_send_sems.at[0],
                recv_sem=p_recv_sems.at[j],
                device_id=(j,), device_id_type=pl.DeviceIdType.MESH,
            )
            recv.wait_recv()
            pj = p_in[pl.ds(j, 1)][0]
            ctx_acc = ctx_acc + pj[:, :S, :].astype(jnp.float32)
            s_acc = s_acc + pj[:, S:, :].astype(jnp.float32)
            return ctx_acc, s_acc

        ctx_acc, s_acc = lax.fori_loop(0, my, combine, (ctx_acc0, s_acc0))

        for b in range(B):
            cols = []
            for h in range(Hq):
                hs = slice(h * Dh, (h + 1) * Dh)
                cols.append(ctx_acc[b][:, hs] / s_acc[b, h][:, None])
            ctx = jnp.concatenate(cols, axis=1).astype(bf)
            out_ref[b] = jnp.dot(ctx, wo,
                                 preferred_element_type=jnp.float32)

        for jj in range(N_DEV - 1):
            @pl.when(jj < my)
            def _():
                d = pltpu.make_async_remote_copy(
                    src_ref=q_buf.at[my], dst_ref=q_buf.at[my],
                    send_sem=q_send_sems.at[jj],
                    recv_sem=q_recv_sems.at[0],
                    device_id=(jj,), device_id_type=pl.DeviceIdType.MESH,
                )
                d.wait_send()

        def drain_p(i, carry):
            d = pltpu.make_async_remote_copy(
                src_ref=p_out.at[i], dst_ref=p_in.at[my],
                send_sem=p_send_sems.at[i],
                recv_sem=p_recv_sems.at[0],
                device_id=(i,), device_id_type=pl.DeviceIdType.MESH,
            )
            d.wait_send()
            return carry

        lax.fori_loop(my + 1, N_DEV, drain_p, 0)

    return pl.pallas_call(
        body,
        out_shape=jax.ShapeDtypeStruct((B, S, DM), jnp.float32),
        in_specs=[pl.BlockSpec(memory_space=pltpu.VMEM)] * 5,
        out_specs=pl.BlockSpec(memory_space=pltpu.VMEM),
        scratch_shapes=[
            pltpu.VMEM((N_DEV, B, S, HD), jnp.bfloat16),
            pltpu.VMEM((N_DEV, B, SP, HD), jnp.bfloat16),
            pltpu.VMEM((N_DEV, B, SP, HD), jnp.bfloat16),
            pltpu.SemaphoreType.DMA((N_DEV - 1,)),
            pltpu.SemaphoreType.DMA((N_DEV,)),
            pltpu.SemaphoreType.DMA((N_DEV,)),
            pltpu.SemaphoreType.DMA((N_DEV,)),
        ],
        compiler_params=pltpu.CompilerParams(
            vmem_limit_bytes=100 * 1024 * 1024,
        ),
    )(x, Wq, K2, V2, Wo)
